# Initial kernel scaffold; baseline (speedup 1.0000x reference)
import jax
import jax.numpy as jnp
from jax.experimental import pallas as pl


def kernel(x, edge_index, edge_weight, W_gcn, b_gcn, W_tcn, b_tcn):
    raise NotImplementedError("write your pallas kernel here")


# speedup vs baseline: 1.5888x; 1.5888x over previous
import functools

import jax
import jax.numpy as jnp
from jax import lax
from jax.experimental import pallas as pl
from jax.experimental.pallas import tpu as pltpu
from jax.experimental.pallas import tpu_sc as plsc

B, IN_F, C, T = 4, 64, 10000, 12
OUT_F = 64
E = 80000
K = 3

NW = 32
R = 320
C_PAD = NW * R
DC = B * OUT_F
NCHUNK = T
EBLK = 2000
NBLK = E // EBLK
GB = 64
NTOT = float(C * T)
EPS = 1e-5

_mesh = plsc.VectorSubcoreMesh(core_axis_name="c", subcore_axis_name="s")
_sc_params = pltpu.CompilerParams(needs_layout_passes=False)


@functools.partial(
    pl.kernel,
    mesh=_mesh,
    compiler_params=_sc_params,
    out_type=jax.ShapeDtypeStruct((C_PAD,), jnp.float32),
    scratch_types=[
        pltpu.VMEM((EBLK,), jnp.int32),
        pltpu.VMEM((EBLK,), jnp.float32),
        pltpu.VMEM((R * 16,), jnp.float32),
        pltpu.VMEM((R,), jnp.float32),
    ],
)
def _deg_kernel(dst_hbm, w_hbm, deg_hbm, dstS, wS, dacc, degL):
    wid = lax.axis_index("s") * 2 + lax.axis_index("c")
    lo = wid * R
    iota = lax.iota(jnp.int32, 16)

    def zero(i, carry):
        dacc[pl.ds(i * 16, 16)] = jnp.zeros((16,), jnp.float32)
        return carry

    lax.fori_loop(0, R, zero, 0)

    def blk(b, carry):
        base = b * EBLK
        pltpu.sync_copy(dst_hbm.at[pl.ds(base, EBLK)], dstS)
        pltpu.sync_copy(w_hbm.at[pl.ds(base, EBLK)], wS)

        def grp(g, c2):
            d16 = dstS[pl.ds(g * 16, 16)]
            w16 = wS[pl.ds(g * 16, 16)]
            m = (d16 >= lo) & (d16 < lo + R)
            ld = jnp.where(m, d16 - lo, 0)
            plsc.addupdate_scatter(dacc, [ld * 16 + iota], w16, mask=m)
            return c2

        return lax.fori_loop(0, EBLK // 16, grp, carry)

    lax.fori_loop(0, NBLK, blk, 0)

    def red(g, carry):
        s = jnp.zeros((16,), jnp.float32)
        for col in range(16):
            s = s + plsc.load_gather(dacc, [(g * 16 + iota) * 16 + col])
        degL[pl.ds(g * 16, 16)] = 1.0 + s
        return carry

    lax.fori_loop(0, R // 16, red, 0)
    pltpu.sync_copy(degL, deg_hbm.at[pl.ds(lo, R)])


@functools.partial(
    pl.kernel,
    mesh=_mesh,
    compiler_params=_sc_params,
    out_type=jax.ShapeDtypeStruct((NCHUNK * C_PAD, DC), jnp.float32),
    scratch_types=[
        pltpu.VMEM((R, DC), jnp.float32),
        pltpu.VMEM((EBLK,), jnp.int32),
        pltpu.VMEM((EBLK,), jnp.int32),
        pltpu.VMEM((EBLK,), jnp.float32),
        pltpu.VMEM((EBLK + GB,), jnp.int32),
        pltpu.VMEM((EBLK + GB,), jnp.int32),
        pltpu.VMEM((EBLK + GB,), jnp.float32),
        pltpu.VMEM((C_PAD,), jnp.float32),
        pltpu.VMEM((GB, DC), jnp.float32),
        pltpu.VMEM((GB,), jnp.int32),
        pltpu.SemaphoreType.DMA,
    ],
)
def _agg_kernel(src_hbm, dst_hbm, w_hbm, dinv_hbm, xw_hbm, agg_hbm,
                acc, dstS, srcS, wS, Lsrc, Lbase, Lnorm, dinvL, rows, gidx,
                sem):
    wid = lax.axis_index("s") * 2 + lax.axis_index("c")
    lo = wid * R
    iota = lax.iota(jnp.int32, 16)
    pltpu.sync_copy(dinv_hbm, dinvL)

    def do_batches(j, cnt):
        def padg(i, carry):
            z16 = jnp.zeros((16,), jnp.int32)
            Lsrc[pl.ds(cnt + i * 16, 16)] = z16
            Lbase[pl.ds(cnt + i * 16, 16)] = z16
            Lnorm[pl.ds(cnt + i * 16, 16)] = jnp.zeros((16,), jnp.float32)
            return carry

        lax.fori_loop(0, GB // 16, padg, 0)
        nb = (cnt + GB - 1) // GB

        def batch(ib, carry):
            eb = ib * GB
            for q in range(GB // 16):
                gidx[pl.ds(q * 16, 16)] = (
                    Lsrc[pl.ds(eb + q * 16, 16)] + j * C_PAD)
            pltpu.async_copy(xw_hbm.at[gidx], rows, sem).wait()

            def edge(e, c4):
                sp = jnp.full((16,), eb + e, jnp.int32)
                nsp = plsc.load_gather(Lnorm, [sp])
                bsp = plsc.load_gather(Lbase, [sp])
                for k in range(DC // 16):
                    v = rows[e, pl.ds(k * 16, 16)]
                    plsc.addupdate_scatter(
                        acc, [bsp, iota + (k * 16)], v * nsp)
                return c4

            lax.fori_loop(0, GB, edge, 0)
            return carry

        lax.fori_loop(0, nb, batch, 0)

    def chunk_body(j, carry):
        def zero(r, c2):
            for k in range(DC // 16):
                acc[r, pl.ds(k * 16, 16)] = jnp.zeros((16,), jnp.float32)
            return c2

        lax.fori_loop(0, R, zero, 0)

        def blk(b, c2):
            base = b * EBLK
            pltpu.sync_copy(dst_hbm.at[pl.ds(base, EBLK)], dstS)
            pltpu.sync_copy(src_hbm.at[pl.ds(base, EBLK)], srcS)
            pltpu.sync_copy(w_hbm.at[pl.ds(base, EBLK)], wS)

            def grp(g, cn):
                d16 = dstS[pl.ds(g * 16, 16)]
                s16 = srcS[pl.ds(g * 16, 16)]
                w16 = wS[pl.ds(g * 16, 16)]
                m = (d16 >= lo) & (d16 < lo + R)
                nrm = (plsc.load_gather(dinvL, [s16]) * w16
                       * plsc.load_gather(dinvL, [d16]))
                plsc.store_compressed(Lsrc.at[pl.ds(cn, 16)], s16, mask=m)
                plsc.store_compressed(
                    Lbase.at[pl.ds(cn, 16)], d16 - lo, mask=m)
                plsc.store_compressed(Lnorm.at[pl.ds(cn, 16)], nrm, mask=m)
                return cn + jnp.sum(m.astype(jnp.int32))

            cnt = lax.fori_loop(0, EBLK // 16, grp, 0)
            do_batches(j, cnt)
            return c2

        lax.fori_loop(0, NBLK, blk, 0)

        def slg(g, c2):
            idx16 = lo + g * 16 + iota
            dv = plsc.load_gather(dinvL, [idx16])
            Lsrc[pl.ds(g * 16, 16)] = idx16
            Lbase[pl.ds(g * 16, 16)] = g * 16 + iota
            Lnorm[pl.ds(g * 16, 16)] = dv * dv
            return c2

        lax.fori_loop(0, R // 16, slg, 0)
        do_batches(j, R)
        pltpu.sync_copy(acc, agg_hbm.at[pl.ds(j * C_PAD + lo, R)])
        return carry

    lax.fori_loop(0, NCHUNK, chunk_body, 0)


CB = 200
NCB = C // CB
CBX = 40
NCBX = C // CBX


def _tc1_body(x_ref, w_ref, deg_ref, xw_ref, dinv_ref):
    w = w_ref[...]
    for b in range(B):
        xb = x_ref[b]
        xt = jnp.transpose(xb, (2, 1, 0))
        r = jnp.dot(xt.reshape(T * CBX, IN_F), w,
                    preferred_element_type=jnp.float32)
        xw_ref[:, :, b * OUT_F:(b + 1) * OUT_F] = r.reshape(NCHUNK, CBX, OUT_F)

    @pl.when(pl.program_id(0) == 0)
    def _():
        dg = deg_ref[...]
        dinv_ref[...] = jnp.where(dg > 0, lax.rsqrt(dg), 0.0)


def _tc2_body(a_ref, s_ref):
    blk = a_ref[...]
    s = jnp.sum(blk, axis=(0, 1))
    s2 = jnp.sum(blk * blk, axis=(0, 1))

    @pl.when(pl.program_id(0) == 0)
    def _():
        s_ref[...] = jnp.zeros((2, DC), jnp.float32)

    s_ref[...] += jnp.stack([s, s2])


def _tc3_body(a_ref, s1_ref, wt_ref, h2_ref, s2_ref):
    s1 = s1_ref[...]
    mu = s1[0] / NTOT
    var = s1[1] / NTOT - mu * mu
    inv = lax.rsqrt(var + EPS)
    a = a_ref[...]
    h1 = jnp.maximum((a - mu) * inv, 0.0)
    zpad = jnp.zeros((1, CB, DC), jnp.float32)
    h1p = jnp.concatenate([zpad, h1, zpad], axis=0)
    s_parts = []
    s2_parts = []
    for b in range(B):
        out = jnp.zeros((T * CB, OUT_F), jnp.float32)
        for k in range(K):
            seg = h1p[k:k + T, :, b * OUT_F:(b + 1) * OUT_F]
            out = out + jnp.dot(seg.reshape(T * CB, OUT_F), wt_ref[k],
                                preferred_element_type=jnp.float32)
        h2b = out.reshape(NCHUNK, CB, OUT_F)
        h2_ref[:, :, b * OUT_F:(b + 1) * OUT_F] = h2b
        s_parts.append(jnp.sum(h2b, axis=(0, 1)))
        s2_parts.append(jnp.sum(h2b * h2b, axis=(0, 1)))
    s = jnp.concatenate(s_parts)
    s2 = jnp.concatenate(s2_parts)

    @pl.when(pl.program_id(0) == 0)
    def _():
        s2_ref[...] = jnp.zeros((2, DC), jnp.float32)

    s2_ref[...] += jnp.stack([s, s2])


def _tc4_body(h2_ref, s2_ref, x_ref, o_ref):
    s2 = s2_ref[...]
    mu = s2[0] / NTOT
    var = s2[1] / NTOT - mu * mu
    inv = lax.rsqrt(var + EPS)
    h = jnp.maximum((h2_ref[...] - mu) * inv, 0.0)
    for b in range(B):
        hb = h[:, :, b * OUT_F:(b + 1) * OUT_F]
        ht = jnp.transpose(hb, (2, 1, 0))
        o_ref[b] = jnp.maximum(ht + x_ref[b], 0.0)


def _tc1_call(x, w_gcn, deg):
    return pl.pallas_call(
        _tc1_body,
        grid=(NCBX,),
        in_specs=[
            pl.BlockSpec((B, IN_F, CBX, T), lambda i: (0, 0, i, 0)),
            pl.BlockSpec((IN_F, OUT_F), lambda i: (0, 0)),
            pl.BlockSpec((C_PAD,), lambda i: (0,)),
        ],
        out_specs=[
            pl.BlockSpec((NCHUNK, CBX, DC), lambda i: (0, i, 0)),
            pl.BlockSpec((C_PAD,), lambda i: (0,)),
        ],
        out_shape=[
            jax.ShapeDtypeStruct((NCHUNK, C_PAD, DC), jnp.float32),
            jax.ShapeDtypeStruct((C_PAD,), jnp.float32),
        ],
    )(x, w_gcn, deg)


def _tc2_call(agg):
    return pl.pallas_call(
        _tc2_body,
        grid=(NCB,),
        in_specs=[pl.BlockSpec((NCHUNK, CB, DC), lambda i: (0, i, 0))],
        out_specs=pl.BlockSpec((2, DC), lambda i: (0, 0)),
        out_shape=jax.ShapeDtypeStruct((2, DC), jnp.float32),
    )(agg)


def _tc3_call(agg, s1, wt):
    return pl.pallas_call(
        _tc3_body,
        grid=(NCB,),
        in_specs=[
            pl.BlockSpec((NCHUNK, CB, DC), lambda i: (0, i, 0)),
            pl.BlockSpec((2, DC), lambda i: (0, 0)),
            pl.BlockSpec((K, OUT_F, OUT_F), lambda i: (0, 0, 0)),
        ],
        out_specs=[
            pl.BlockSpec((NCHUNK, CB, DC), lambda i: (0, i, 0)),
            pl.BlockSpec((2, DC), lambda i: (0, 0)),
        ],
        out_shape=[
            jax.ShapeDtypeStruct((NCHUNK, C_PAD, DC), jnp.float32),
            jax.ShapeDtypeStruct((2, DC), jnp.float32),
        ],
    )(agg, s1, wt)


def _tc4_call(h2, s2, x):
    return pl.pallas_call(
        _tc4_body,
        grid=(NCBX,),
        in_specs=[
            pl.BlockSpec((NCHUNK, CBX, DC), lambda i: (0, i, 0)),
            pl.BlockSpec((2, DC), lambda i: (0, 0)),
            pl.BlockSpec((B, IN_F, CBX, T), lambda i: (0, 0, i, 0)),
        ],
        out_specs=pl.BlockSpec((B, IN_F, CBX, T), lambda i: (0, 0, i, 0)),
        out_shape=jax.ShapeDtypeStruct((B, IN_F, C, T), jnp.float32),
    )(h2, s2, x)


def kernel(x, edge_index, edge_weight, W_gcn, b_gcn, W_tcn, b_tcn):
    del b_gcn, b_tcn
    src = edge_index[0]
    dst = edge_index[1]
    w = edge_weight.astype(jnp.float32)

    deg = _deg_kernel(dst, w)
    xw, dinv = _tc1_call(x, W_gcn, deg)
    xw_flat = xw.reshape(NCHUNK * C_PAD, DC)
    agg_flat = _agg_kernel(src, dst, w, dinv, xw_flat)
    agg = agg_flat.reshape(NCHUNK, C_PAD, DC)
    s1 = _tc2_call(agg)
    wt = jnp.transpose(W_tcn[:, :, 0, :], (2, 1, 0))
    h2, s2 = _tc3_call(agg, s1, wt)
    return _tc4_call(h2, s2, x)
